# baseline (speedup 1.0000x reference)
import numpy as np
import jax
import jax.numpy as jnp
from jax.experimental import pallas as pl
from jax.experimental.pallas import tpu as pltpu

B = 128
NBO = 8
BS = 256
NINP = 1024
GH = 3 * BS
NH = 4
DHID = NBO * BS
TOPK = 4

BF = jnp.bfloat16
F32 = jnp.float32


def _attn_consts():
    seg = np.zeros((NBO * 64, NBO * NH), np.float32)
    for j in range(NBO):
        for h in range(NH):
            seg[j * 64 + h * 16: j * 64 + h * 16 + 16, j * NH + h] = 0.25
    g = np.zeros((NBO * NH, NBO * NH), np.float32)
    for c in range(NBO * NH):
        for c2 in range(NBO * NH):
            if c % NH == c2 % NH:
                g[c, c2] = 1.0
    ebig = np.zeros((NBO * NH, NBO * 64), np.float32)
    for j in range(NBO):
        for h in range(NH):
            ebig[j * NH + h, j * 64 + h * 16: j * 64 + h * 16 + 16] = 1.0
    f = np.zeros((NBO * 64, 64), np.float32)
    for j in range(NBO):
        f[j * 64:(j + 1) * 64, :] = np.eye(64, dtype=np.float32)
    return seg, g, ebig, f


_SEG, _G, _EBIG, _F = _attn_consts()


def _dot(a, b):
    return jax.lax.dot(a.astype(BF), b.astype(BF),
                       preferred_element_type=F32)


def _b(x):
    return x.astype(BF).astype(F32)


def _gru_core(inp_ref, hx_ref, hxb_ref, ia_wq_ref, ia_wk_ref, ia_wv_ref,
              wi_ref, wh_ref, bi_ref, bh_ref,
              hn_ref, m8_ref, v1_s, p1_s):
    j = pl.program_id(0)

    @pl.when(j == 0)
    def _scores():
        inp = inp_ref[...]
        hx = hx_ref[...]
        k1 = _dot(inp, ia_wk_ref[0])
        v1_s[...] = _dot(inp, ia_wv_ref[0])
        ljs = []
        for jj in range(NBO):
            hbj = hx[:, jj * BS:(jj + 1) * BS]
            qj = _dot(hbj, ia_wq_ref[jj])
            ljs.append(jnp.sum(_b(qj) * _b(k1), axis=1, keepdims=True)
                       * 0.125)
        logits = jnp.concatenate(ljs, axis=1)
        col = jax.lax.broadcasted_iota(jnp.int32, (B, NBO), 1)
        mcols = []
        for jj in range(NBO):
            lj = ljs[jj]
            below = (logits < lj) | ((logits == lj) & (col < jj))
            cnt = jnp.sum(below.astype(F32), axis=1, keepdims=True)
            mcols.append((cnt >= TOPK).astype(F32))
            p1_s[jj] = jax.nn.sigmoid(lj)
        m8_ref[...] = jnp.concatenate(mcols, axis=1)

    hbj = hxb_ref[...]
    gi = p1_s[j] * _dot(v1_s[...], wi_ref[0]) + bi_ref[0]
    gh = _dot(hbj, wh_ref[0]) + bh_ref[0]
    r = jax.nn.sigmoid(gi[:, :BS] + gh[:, :BS])
    z = jax.nn.sigmoid(gi[:, BS:2 * BS] + gh[:, BS:2 * BS])
    n = jnp.tanh(gi[:, 2 * BS:] + r * gh[:, 2 * BS:])
    hn_ref[...] = (1.0 - z) * n + z * hbj


def _attn_core(hn_ref, hx_ref, cx_ref, m8_ref, mwq_ref, mwk_ref, mwv_ref,
               wfc_ref, bfc_ref, wg_ref, bg_ref,
               seg_ref, g_ref, ebig_ref, f_ref,
               hx_out_ref, cx_out_ref, mask_out_ref):
    hns = [hn_ref[:, jj * BS:(jj + 1) * BS] for jj in range(NBO)]
    qs = [_dot(hns[jj], mwq_ref[jj]) for jj in range(NBO)]
    kb = _b(jnp.concatenate(
        [_dot(hns[jj], mwk_ref[jj]) for jj in range(NBO)], axis=1))
    vb = _b(jnp.concatenate(
        [_dot(hns[jj], mwv_ref[jj]) for jj in range(NBO)], axis=1))
    seg = seg_ref[...]
    gmat = g_ref[...]
    ebig = ebig_ref[...]
    fmat = f_ref[...]
    wfc = wfc_ref[...]
    wg = wg_ref[...]
    bfc = bfc_ref[...]
    bg = bg_ref[...]
    hx = hx_ref[...]
    cx = cx_ref[...]
    m8 = m8_ref[...]
    for i in range(NBO):
        qt = jnp.concatenate([qs[i]] * NBO, axis=1)
        s = _dot(_b(qt) * kb, seg)
        e = jnp.exp(s)
        pn = e / _dot(e, gmat)
        w = _dot(pn, ebig)
        out = _dot(_b(w) * vb, fmat)
        o = _dot(out, wfc) + bfc
        a = _dot(out, wg) + bg
        hfin = hns[i] + jax.nn.sigmoid(a) * jnp.tanh(o)
        m = m8[:, i:i + 1]
        sl = slice(i * BS, (i + 1) * BS)
        hx_out_ref[:, sl] = m * hfin + (1.0 - m) * hx[:, sl]
        cx_out_ref[:, sl] = m * hns[i] + (1.0 - m) * cx[:, sl]
        mask_out_ref[:, sl] = jnp.broadcast_to(m, (B, BS))


def kernel(inp, hx, cx, ia_wq, ia_wk, ia_wv, mha_wq, mha_wk, mha_wv,
           mha_wfc, mha_bfc, mha_wg, mha_bg, gru_wi, gru_wh, gru_bi,
           gru_bh, step):
    const = lambda shape: pl.BlockSpec(shape, lambda j: (0,) * len(shape))
    hn, m8 = pl.pallas_call(
        _gru_core,
        grid=(NBO,),
        in_specs=[
            const((B, NINP)),
            const((B, DHID)),
            pl.BlockSpec((B, BS), lambda j: (0, j)),
            const((NBO, BS, 64)),
            pl.BlockSpec((1, NINP, 64), lambda j: (1, 0, 0)),
            pl.BlockSpec((1, NINP, NINP), lambda j: (1, 0, 0)),
            pl.BlockSpec((1, NINP, GH), lambda j: (j, 0, 0)),
            pl.BlockSpec((1, BS, GH), lambda j: (j, 0, 0)),
            pl.BlockSpec((1, 1, GH), lambda j: (j, 0, 0)),
            pl.BlockSpec((1, 1, GH), lambda j: (j, 0, 0)),
        ],
        out_specs=[pl.BlockSpec((B, BS), lambda j: (0, j)),
                   const((B, NBO))],
        out_shape=[jax.ShapeDtypeStruct((B, DHID), F32),
                   jax.ShapeDtypeStruct((B, NBO), F32)],
        scratch_shapes=[
            pltpu.VMEM((B, NINP), F32),
            pltpu.VMEM((NBO, B, 1), F32),
        ],
        compiler_params=pltpu.CompilerParams(
            dimension_semantics=("arbitrary",)),
    )(inp, hx, hx, ia_wq, ia_wk, ia_wv, gru_wi, gru_wh,
      gru_bi.reshape(NBO, 1, GH), gru_bh.reshape(NBO, 1, GH))

    out_shape = [jax.ShapeDtypeStruct((B, DHID), F32) for _ in range(3)]
    hx_out, cx_out, mask = pl.pallas_call(
        _attn_core,
        out_shape=out_shape,
    )(hn, hx, cx, m8, mha_wq, mha_wk, mha_wv, mha_wfc,
      mha_bfc.reshape(1, BS), mha_wg, mha_bg.reshape(1, BS),
      jnp.asarray(_SEG), jnp.asarray(_G), jnp.asarray(_EBIG),
      jnp.asarray(_F))
    return hx_out, cx_out, mask


# speedup vs baseline: 1.0587x; 1.0587x over previous
import numpy as np
import jax
import jax.numpy as jnp
from jax.experimental import pallas as pl
from jax.experimental.pallas import tpu as pltpu

B = 128
NBO = 8
BS = 256
NINP = 1024
GH = 3 * BS
NH = 4
DK = 16
DHID = NBO * BS
TOPK = 4

BF = jnp.bfloat16


def _attn_consts():
    seg = np.zeros((NBO * 64, NBO * NH), np.float32)
    for j in range(NBO):
        for h in range(NH):
            seg[j * 64 + h * 16: j * 64 + h * 16 + 16, j * NH + h] = 0.25
    g = np.zeros((NBO * NH, NBO * NH), np.float32)
    for c in range(NBO * NH):
        for c2 in range(NBO * NH):
            if c % NH == c2 % NH:
                g[c, c2] = 1.0
    ebig = np.zeros((NBO * NH, NBO * 64), np.float32)
    for j in range(NBO):
        for h in range(NH):
            ebig[j * NH + h, j * 64 + h * 16: j * 64 + h * 16 + 16] = 1.0
    f = np.zeros((NBO * 64, 64), np.float32)
    for j in range(NBO):
        f[j * 64:(j + 1) * 64, :] = np.eye(64, dtype=np.float32)
    return seg, g, ebig, f


_SEG, _G, _EBIG, _F = _attn_consts()


def _dot(a, b):
    return jax.lax.dot(a, b, preferred_element_type=jnp.float32)


def _b(x):
    return x.astype(BF).astype(jnp.float32)


def _core(inp_ref, hx_ref, cx_ref, ia_wq_ref, ia_wk_ref, ia_wv_ref,
          mwq_ref, mwk_ref, mwv_ref, wfc_ref, bfc_ref, wg_ref, bg_ref,
          wi_ref, wh_ref, bi_ref, bh_ref,
          seg_ref, g_ref, ebig_ref, f_ref,
          hx_out_ref, cx_out_ref, mask_out_ref):
    inp = inp_ref[...]
    hx = hx_ref[...]

    k1 = _dot(inp, ia_wk_ref[1])
    v1 = _dot(inp, ia_wv_ref[1])

    ljs = []
    for j in range(NBO):
        hbj = hx[:, j * BS:(j + 1) * BS]
        qj = _dot(hbj, ia_wq_ref[j])
        ljs.append(jnp.sum(_b(qj) * _b(k1), axis=1, keepdims=True) * 0.125)
    logits = jnp.concatenate(ljs, axis=1)

    col = jax.lax.broadcasted_iota(jnp.int32, (B, NBO), 1)
    masks, p1s = [], []
    for j in range(NBO):
        lj = ljs[j]
        below = (logits < lj) | ((logits == lj) & (col < j))
        cnt = jnp.sum(below.astype(jnp.float32), axis=1, keepdims=True)
        masks.append((cnt >= TOPK).astype(jnp.float32))
        p1s.append(jax.nn.sigmoid(lj))

    hns = []
    for j in range(NBO):
        hbj = hx[:, j * BS:(j + 1) * BS]
        gi = p1s[j] * _dot(v1, wi_ref[j]) + bi_ref[j:j + 1, :]
        gh = _dot(hbj, wh_ref[j]) + bh_ref[j:j + 1, :]
        r = jax.nn.sigmoid(gi[:, :BS] + gh[:, :BS])
        z = jax.nn.sigmoid(gi[:, BS:2 * BS] + gh[:, BS:2 * BS])
        n = jnp.tanh(gi[:, 2 * BS:] + r * gh[:, 2 * BS:])
        hns.append((1.0 - z) * n + z * hbj)

    qs = [_dot(hns[j], mwq_ref[j]) for j in range(NBO)]
    kcat = jnp.concatenate([_dot(hns[j], mwk_ref[j]) for j in range(NBO)],
                           axis=1)
    vcat = jnp.concatenate([_dot(hns[j], mwv_ref[j]) for j in range(NBO)],
                           axis=1)
    seg = seg_ref[...]
    gmat = g_ref[...]
    ebig = ebig_ref[...]
    fmat = f_ref[...]
    wfc = wfc_ref[...]
    wg = wg_ref[...]
    bfc = bfc_ref[...]
    bg = bg_ref[...]
    hfin = []
    for i in range(NBO):
        qt = jnp.concatenate([qs[i]] * NBO, axis=1)
        s = _dot(_b(qt) * _b(kcat), seg)
        e = jnp.exp(s)
        pn = e / _dot(e, gmat)
        w = _dot(pn, ebig)
        out = _dot(_b(w) * _b(vcat), fmat)
        o = _dot(out, wfc) + bfc
        a = _dot(out, wg) + bg
        hfin.append(hns[i] + jax.nn.sigmoid(a) * jnp.tanh(o))

    cx = cx_ref[...]
    for j in range(NBO):
        m = masks[j]
        sl = slice(j * BS, (j + 1) * BS)
        hx_out_ref[:, sl] = m * hfin[j] + (1.0 - m) * hx[:, sl]
        cx_out_ref[:, sl] = m * hns[j] + (1.0 - m) * cx[:, sl]
        mask_out_ref[:, sl] = jnp.broadcast_to(m, (B, BS))


def kernel(inp, hx, cx, ia_wq, ia_wk, ia_wv, mha_wq, mha_wk, mha_wv,
           mha_wfc, mha_bfc, mha_wg, mha_bg, gru_wi, gru_wh, gru_bi,
           gru_bh, step):
    f32 = jnp.float32
    out_shape = [jax.ShapeDtypeStruct((B, DHID), f32) for _ in range(3)]
    hx_out, cx_out, mask = pl.pallas_call(
        _core,
        out_shape=out_shape,
    )(inp, hx, cx, ia_wq, ia_wk, ia_wv,
      mha_wq, mha_wk, mha_wv, mha_wfc, mha_bfc.reshape(1, BS),
      mha_wg, mha_bg.reshape(1, BS),
      gru_wi, gru_wh, gru_bi, gru_bh,
      jnp.asarray(_SEG), jnp.asarray(_G), jnp.asarray(_EBIG),
      jnp.asarray(_F))
    return hx_out, cx_out, mask


# speedup vs baseline: 1.8266x; 1.7254x over previous
import numpy as np
import jax
import jax.numpy as jnp
from jax.experimental import pallas as pl
from jax.experimental.pallas import tpu as pltpu

B = 128
NBO = 8
BS = 256
NINP = 1024
GH = 3 * BS
NH = 4
DHID = NBO * BS
TOPK = 4

BF = jnp.bfloat16
F32 = jnp.float32


def _attn_consts():
    seg = np.zeros((NBO * 64, NBO * NH), np.float32)
    for j in range(NBO):
        for h in range(NH):
            seg[j * 64 + h * 16: j * 64 + h * 16 + 16, j * NH + h] = 0.25
    g = np.zeros((NBO * NH, NBO * NH), np.float32)
    for c in range(NBO * NH):
        for c2 in range(NBO * NH):
            if c % NH == c2 % NH:
                g[c, c2] = 1.0
    ebig = np.zeros((NBO * NH, NBO * 64), np.float32)
    for j in range(NBO):
        for h in range(NH):
            ebig[j * NH + h, j * 64 + h * 16: j * 64 + h * 16 + 16] = 1.0
    f = np.zeros((NBO * 64, 64), np.float32)
    for j in range(NBO):
        f[j * 64:(j + 1) * 64, :] = np.eye(64, dtype=np.float32)
    return seg, g, ebig, f


_SEG, _G, _EBIG, _F = _attn_consts()


def _dot(a, b):
    return jax.lax.dot(a, b, preferred_element_type=F32)


def _dott(a, bt):
    return jax.lax.dot_general(a, bt, (((1,), (1,)), ((), ())),
                               preferred_element_type=F32)


def _b(x):
    return x.astype(BF).astype(F32)


def _core(inp_ref, hx_ref, cx_ref, wqt_ref, wk1t_ref, ia_wv_ref,
          mwqt_ref, mwkt_ref, mwvt_ref, wfc_ref, bfc_ref, wg_ref, bg_ref,
          wi_hbm, wh_hbm, bi_ref, bh_ref,
          seg_ref, g_ref, ebig_ref, f_ref,
          hx_out_ref, cx_out_ref, mask_out_ref,
          wi_s, wh_s, sem_i, sem_h):
    for j in range(NBO):
        pltpu.make_async_copy(wi_hbm.at[j], wi_s.at[j], sem_i.at[j]).start()
        pltpu.make_async_copy(wh_hbm.at[j], wh_s.at[j], sem_h.at[j]).start()

    inp = inp_ref[...]
    hx = hx_ref[...]

    k1 = _dott(inp, wk1t_ref[0])
    v1 = _dot(inp, ia_wv_ref[0])

    ljs = []
    for j in range(NBO):
        hbj = hx[:, j * BS:(j + 1) * BS]
        qj = _dott(hbj, wqt_ref[j])
        ljs.append(jnp.sum(_b(qj) * _b(k1), axis=1, keepdims=True) * 0.125)
    logits = jnp.concatenate(ljs, axis=1)

    col = jax.lax.broadcasted_iota(jnp.int32, (B, NBO), 1)
    masks, p1s = [], []
    for j in range(NBO):
        lj = ljs[j]
        below = (logits < lj) | ((logits == lj) & (col < j))
        cnt = jnp.sum(below.astype(F32), axis=1, keepdims=True)
        masks.append((cnt >= TOPK).astype(F32))
        p1s.append(jax.nn.sigmoid(lj))

    hns = []
    for j in range(NBO):
        pltpu.make_async_copy(wi_hbm.at[j], wi_s.at[j], sem_i.at[j]).wait()
        pltpu.make_async_copy(wh_hbm.at[j], wh_s.at[j], sem_h.at[j]).wait()
        hbj = hx[:, j * BS:(j + 1) * BS]
        gi = p1s[j] * _dot(v1, wi_s[j]) + bi_ref[j:j + 1, :]
        gh = _dot(hbj, wh_s[j]) + bh_ref[j:j + 1, :]
        r = jax.nn.sigmoid(gi[:, :BS] + gh[:, :BS])
        z = jax.nn.sigmoid(gi[:, BS:2 * BS] + gh[:, BS:2 * BS])
        n = jnp.tanh(gi[:, 2 * BS:] + r * gh[:, 2 * BS:])
        hns.append((1.0 - z) * n + z * hbj)

    qs = [_dott(hns[j], mwqt_ref[j]) for j in range(NBO)]
    kcat = jnp.concatenate(
        [_dott(hns[j], mwkt_ref[j]) for j in range(NBO)], axis=1)
    vcat = jnp.concatenate(
        [_dott(hns[j], mwvt_ref[j]) for j in range(NBO)], axis=1)
    seg = seg_ref[...]
    gmat = g_ref[...]
    ebig = ebig_ref[...]
    fmat = f_ref[...]
    wfc = wfc_ref[...]
    wg = wg_ref[...]
    bfc = bfc_ref[...]
    bg = bg_ref[...]
    hfin = []
    for i in range(NBO):
        qt = jnp.concatenate([qs[i]] * NBO, axis=1)
        s = _dot(_b(qt) * _b(kcat), seg)
        e = jnp.exp(s)
        pn = e / _dot(e, gmat)
        w = _dot(pn, ebig)
        out = _dot(_b(w) * _b(vcat), fmat)
        o = _dot(out, wfc) + bfc
        a = _dot(out, wg) + bg
        hfin.append(hns[i] + jax.nn.sigmoid(a) * jnp.tanh(o))

    cx = cx_ref[...]
    for j in range(NBO):
        m = masks[j]
        sl = slice(j * BS, (j + 1) * BS)
        hx_out_ref[:, sl] = m * hfin[j] + (1.0 - m) * hx[:, sl]
        cx_out_ref[:, sl] = m * hns[j] + (1.0 - m) * cx[:, sl]
        mask_out_ref[:, sl] = jnp.broadcast_to(m, (B, BS))


def kernel(inp, hx, cx, ia_wq, ia_wk, ia_wv, mha_wq, mha_wk, mha_wv,
           mha_wfc, mha_bfc, mha_wg, mha_bg, gru_wi, gru_wh, gru_bi,
           gru_bh, step):
    vmem = pl.BlockSpec(memory_space=pltpu.VMEM)
    out_shape = [jax.ShapeDtypeStruct((B, DHID), F32) for _ in range(3)]
    hx_out, cx_out, mask = pl.pallas_call(
        _core,
        grid=(1,),
        in_specs=[
            vmem, vmem, vmem,
            vmem,
            pl.BlockSpec((1, 64, NINP), lambda i: (1, 0, 0)),
            pl.BlockSpec((1, NINP, NINP), lambda i: (1, 0, 0)),
            vmem, vmem, vmem,
            vmem, vmem, vmem, vmem,
            pl.BlockSpec(memory_space=pl.ANY),
            pl.BlockSpec(memory_space=pl.ANY),
            vmem, vmem,
            vmem, vmem, vmem, vmem,
        ],
        out_specs=[vmem] * 3,
        out_shape=out_shape,
        scratch_shapes=[
            pltpu.VMEM((NBO, NINP, GH), F32),
            pltpu.VMEM((NBO, BS, GH), F32),
            pltpu.SemaphoreType.DMA((NBO,)),
            pltpu.SemaphoreType.DMA((NBO,)),
        ],
    )(inp, hx, cx,
      ia_wq.transpose(0, 2, 1), ia_wk.transpose(0, 2, 1), ia_wv,
      mha_wq.transpose(0, 2, 1), mha_wk.transpose(0, 2, 1),
      mha_wv.transpose(0, 2, 1),
      mha_wfc, mha_bfc.reshape(1, BS), mha_wg, mha_bg.reshape(1, BS),
      gru_wi, gru_wh, gru_bi, gru_bh,
      jnp.asarray(_SEG), jnp.asarray(_G), jnp.asarray(_EBIG),
      jnp.asarray(_F))
    return hx_out, cx_out, mask
